# baseline (speedup 1.0000x reference)
import jax
import jax.numpy as jnp
from jax.experimental import pallas as pl
from jax.experimental.pallas import tpu as pltpu

_NUM_CODES = 1024
_EMBED_DIM = 256
_BETA = 0.25
_BR = 4096


_CK = 512


def _vq_block_kernel(z_ref, ct2_ref, cb_ref, cs_ref, qst_ref, idx_ref, dsum_ref):
    zb = z_ref[...]
    zs = jnp.sum(zb * zb, axis=1, keepdims=True)
    iota_l = jax.lax.broadcasted_iota(
        jnp.int32, (_BR, _CK), 1).astype(jnp.float32)
    m = None
    idx_f = None
    for k in range(_NUM_CODES // _CK):
        sl = pl.ds(k * _CK, _CK)
        mm2 = jnp.dot(zb, ct2_ref[:, sl],
                      preferred_element_type=jnp.float32)
        dk = jnp.sqrt(jnp.maximum(zs + cs_ref[:, sl] - mm2, 0.0))
        mk = jnp.min(dk, axis=1, keepdims=True)
        ik = jnp.min(jnp.where(dk == mk, iota_l, 2048.0), axis=1,
                     keepdims=True) + float(k * _CK)
        if m is None:
            m, idx_f = mk, ik
        else:
            idx_f = jnp.where(mk < m, ik, idx_f)
            m = jnp.minimum(m, mk)
    iota_f = jax.lax.broadcasted_iota(
        jnp.int32, (_BR, _NUM_CODES), 1).astype(jnp.float32)
    onehot = (iota_f == idx_f).astype(jnp.bfloat16)
    q = jax.lax.dot_general(
        onehot, cb_ref[...], (((1,), (0,)), ((), ())),
        preferred_element_type=jnp.float32)
    qst_ref[...] = q
    idx_ref[...] = idx_f.astype(jnp.int32).reshape(1, 1, _BR)
    s = jnp.sum(m * m, keepdims=True).reshape(1, 1, 1)
    dsum_ref[...] = jnp.broadcast_to(s, (1, 1, 128))


def kernel(z, codebook):
    B, D, H, W = z.shape
    n = B * H * W
    nblk = n // _BR
    z_flat = jnp.transpose(z, (0, 2, 3, 1)).reshape(-1, D)
    ct2 = (codebook * 2.0).T
    cs = jnp.sum(codebook * codebook, axis=1)[None, :]
    cb16 = codebook.astype(jnp.bfloat16)
    qst, idx3, dsum = pl.pallas_call(
        _vq_block_kernel,
        grid=(nblk,),
        in_specs=[
            pl.BlockSpec((_BR, D), lambda i: (i, 0)),
            pl.BlockSpec((D, _NUM_CODES), lambda i: (0, 0)),
            pl.BlockSpec((_NUM_CODES, D), lambda i: (0, 0)),
            pl.BlockSpec((1, _NUM_CODES), lambda i: (0, 0)),
        ],
        out_specs=[
            pl.BlockSpec((_BR, D), lambda i: (i, 0)),
            pl.BlockSpec((1, 1, _BR), lambda i: (i, 0, 0)),
            pl.BlockSpec((1, 1, 128), lambda i: (i, 0, 0)),
        ],
        out_shape=[
            jax.ShapeDtypeStruct((n, D), jnp.float32),
            jax.ShapeDtypeStruct((nblk, 1, _BR), jnp.int32),
            jax.ShapeDtypeStruct((nblk, 1, 128), jnp.float32),
        ],
        compiler_params=pltpu.CompilerParams(
            dimension_semantics=("parallel",),
            allow_input_fusion=[True, False, False, False],
            vmem_limit_bytes=60 * 1024 * 1024,
            disable_bounds_checks=True,
        ),
    )(z_flat, ct2, cb16, cs)
    z_q = jnp.transpose(qst.reshape(B, H, W, D), (0, 3, 1, 2))
    vq_loss = (1.0 + _BETA) * (jnp.sum(dsum[:, 0, 0]) / (n * D))
    indices = idx3.reshape(B, H, W)
    return (z_q, vq_loss, indices)


# speedup vs baseline: 1.0207x; 1.0207x over previous
import jax
import jax.numpy as jnp
from jax.experimental import pallas as pl
from jax.experimental.pallas import tpu as pltpu

_NUM_CODES = 1024
_EMBED_DIM = 256
_BETA = 0.25
_BR = 2048


_CK = 512


def _vq_block_kernel(z_ref, ct2_ref, cb_ref, cs_ref, qst_ref, idx_ref, dsum_ref):
    zb = z_ref[...]
    zs = jnp.sum(zb * zb, axis=1, keepdims=True)
    iota_l = jax.lax.broadcasted_iota(
        jnp.int32, (_BR, _CK), 1).astype(jnp.float32)
    m = None
    idx_f = None
    for k in range(_NUM_CODES // _CK):
        sl = pl.ds(k * _CK, _CK)
        mm2 = jnp.dot(zb, ct2_ref[:, sl],
                      preferred_element_type=jnp.float32)
        dk = jnp.sqrt(jnp.maximum(zs + cs_ref[:, sl] - mm2, 0.0))
        mk = jnp.min(dk, axis=1, keepdims=True)
        ik = jnp.min(jnp.where(dk == mk, iota_l, 2048.0), axis=1,
                     keepdims=True) + float(k * _CK)
        if m is None:
            m, idx_f = mk, ik
        else:
            idx_f = jnp.where(mk < m, ik, idx_f)
            m = jnp.minimum(m, mk)
    iota_f = jax.lax.broadcasted_iota(
        jnp.int32, (_BR, _NUM_CODES), 1).astype(jnp.float32)
    onehot = (iota_f == idx_f).astype(jnp.bfloat16)
    q = jax.lax.dot_general(
        onehot, cb_ref[...], (((1,), (0,)), ((), ())),
        preferred_element_type=jnp.float32)
    qst_ref[...] = q
    idx_ref[...] = idx_f.astype(jnp.int32).reshape(1, 1, _BR)
    s = jnp.sum(m * m, keepdims=True).reshape(1, 1, 1)
    dsum_ref[...] = jnp.broadcast_to(s, (1, 1, 128))


def kernel(z, codebook):
    B, D, H, W = z.shape
    n = B * H * W
    nblk = n // _BR
    z_flat = jnp.transpose(z, (0, 2, 3, 1)).reshape(-1, D)
    ct2 = (codebook * 2.0).T
    cs = jnp.sum(codebook * codebook, axis=1)[None, :]
    cb16 = codebook.astype(jnp.bfloat16)
    qst, idx3, dsum = pl.pallas_call(
        _vq_block_kernel,
        grid=(nblk,),
        in_specs=[
            pl.BlockSpec((_BR, D), lambda i: (i, 0)),
            pl.BlockSpec((D, _NUM_CODES), lambda i: (0, 0)),
            pl.BlockSpec((_NUM_CODES, D), lambda i: (0, 0)),
            pl.BlockSpec((1, _NUM_CODES), lambda i: (0, 0)),
        ],
        out_specs=[
            pl.BlockSpec((_BR, D), lambda i: (i, 0)),
            pl.BlockSpec((1, 1, _BR), lambda i: (i, 0, 0)),
            pl.BlockSpec((1, 1, 128), lambda i: (i, 0, 0)),
        ],
        out_shape=[
            jax.ShapeDtypeStruct((n, D), jnp.float32),
            jax.ShapeDtypeStruct((nblk, 1, _BR), jnp.int32),
            jax.ShapeDtypeStruct((nblk, 1, 128), jnp.float32),
        ],
        compiler_params=pltpu.CompilerParams(
            dimension_semantics=("parallel",),
            allow_input_fusion=[True, False, False, False],
            vmem_limit_bytes=60 * 1024 * 1024,
            disable_bounds_checks=True,
        ),
    )(z_flat, ct2, cb16, cs)
    z_q = jnp.transpose(qst.reshape(B, H, W, D), (0, 3, 1, 2))
    vq_loss = (1.0 + _BETA) * (jnp.sum(dsum[:, 0, 0]) / (n * D))
    indices = idx3.reshape(B, H, W)
    return (z_q, vq_loss, indices)
